# Initial kernel scaffold; baseline (speedup 1.0000x reference)
import jax
import jax.numpy as jnp
from jax.experimental import pallas as pl


def kernel(global_idx, acts, sign, edge_index, batch, node_emb, edge_emb, W_act, b_act, We, be, Wpre, bpre, Wpost, bpost, Wlin, blin, gamma, beta, W_fc1, b_fc1, W_out, b_out, prelu_a):
    raise NotImplementedError("write your pallas kernel here")


# speedup vs baseline: 5.8911x; 5.8911x over previous
import functools

import jax
import jax.numpy as jnp
import numpy as np
from jax import lax
from jax.experimental import pallas as pl
from jax.experimental.pallas import tpu as pltpu

_N = 10000
_E = 40000
_P919 = 919
_D = 512
_T = 4
_F = 512
_FO = 128
_EDIM = 50
_NG = 64
_DEG_HIST = np.array([0.0, 500.0, 1000.0, 1500.0, 2000.0, 1800.0, 1200.0, 800.0,
                      500.0, 300.0, 200.0, 100.0, 60.0, 30.0, 10.0])
_AVG_DEG_LOG = float((np.log(np.arange(_DEG_HIST.shape[0]) + 1.0) * _DEG_HIST).sum()
                     / _DEG_HIST.sum())

_HI = jax.lax.Precision.DEFAULT


def _dot(a, b):
    return jax.lax.dot_general(a, b, (((1,), (0,)), ((), ())), precision=_HI,
                               preferred_element_type=jnp.float32)


def _kx_body(xe_ref, acts_ref, wact_ref, bact_ref, o_ref):
    o_ref[...] = (xe_ref[...] + _dot(acts_ref[...], wact_ref[...])
                  + bact_ref[...])


def _k_x(xe, acts, W_act, b_act):
    nb = 1000
    return pl.pallas_call(
        _kx_body,
        grid=(_N // nb,),
        in_specs=[
            pl.BlockSpec((nb, _D), lambda i: (i, 0)),
            pl.BlockSpec((nb, 2), lambda i: (i, 0)),
            pl.BlockSpec((2, _D), lambda i: (0, 0)),
            pl.BlockSpec((1, _D), lambda i: (0, 0)),
        ],
        out_specs=pl.BlockSpec((nb, _D), lambda i: (i, 0)),
        out_shape=jax.ShapeDtypeStruct((_N, _D), jnp.float32),
    )(xe, acts, W_act, b_act)


def _kc2_body(ee_ref, we_ref, be_ref, wpe_ref, o_ref):
    e2 = _dot(ee_ref[...], we_ref[...]) + be_ref[...]
    for t in range(_T):
        o_ref[:, t * _F:(t + 1) * _F] = _dot(e2, wpe_ref[t])


def _k_c2(edge_emb, We_l, be_l, Wpre_e_l):
    return pl.pallas_call(
        _kc2_body,
        out_shape=jax.ShapeDtypeStruct((2, _T * _F), jnp.float32),
    )(edge_emb, We_l, be_l.reshape(1, _F), Wpre_e_l)


def _kpre_body(x_ref, wd_ref, ws_ref, bpre_ref, p_ref, q_ref):
    x = x_ref[...]
    p_ref[...] = _dot(x, wd_ref[...]) + bpre_ref[...]
    q_ref[...] = _dot(x, ws_ref[...])


def _k_pre(x, Wd, Ws, bpre_f):
    nb = 400
    return pl.pallas_call(
        _kpre_body,
        grid=(_N // nb,),
        in_specs=[
            pl.BlockSpec((nb, _D), lambda i: (i, 0)),
            pl.BlockSpec((_D, _T * _F), lambda i: (0, 0)),
            pl.BlockSpec((_D, _T * _F), lambda i: (0, 0)),
            pl.BlockSpec((1, _T * _F), lambda i: (0, 0)),
        ],
        out_specs=[
            pl.BlockSpec((nb, _T * _F), lambda i: (i, 0)),
            pl.BlockSpec((nb, _T * _F), lambda i: (i, 0)),
        ],
        out_shape=[
            jax.ShapeDtypeStruct((_N, _T * _F), jnp.float32),
            jax.ShapeDtypeStruct((_N, _T * _F), jnp.float32),
        ],
    )(x, Wd, Ws, bpre_f)


def _kpost_body(x_ref, p_ref, s1_ref, s2_ref, mn_ref, mx_ref, deg_ref,
                wpost_ref, bpost_ref, wlin_ref, blin_ref,
                y_ref, bnsum_ref, bnssq_ref):
    i = pl.program_id(0)
    deg = deg_ref[...]
    degc = jnp.maximum(deg, 1.0)
    has = deg > 0.0
    K = p_ref[...]
    S1 = s1_ref[...]
    mean = (deg * K + S1) / degc
    sumsq = deg * K * K + 2.0 * K * S1 + s2_ref[...]
    var = jnp.maximum(sumsq / degc - mean * mean, 0.0)
    std = jnp.sqrt(var + 1e-5)
    mn = jnp.where(has, K + mn_ref[...], 0.0)
    mx = jnp.where(has, K + mx_ref[...], 0.0)
    ld = jnp.log(degc + 1.0)
    s2c = ld * (1.0 / _AVG_DEG_LOG)
    s3c = _AVG_DEG_LOG / ld
    x = x_ref[...]
    nb = x.shape[0]
    o_parts = []
    for t in range(_T):
        sl = slice(t * _F, (t + 1) * _F)
        agg = jnp.concatenate([mean[:, sl], mn[:, sl], mx[:, sl], std[:, sl]],
                              axis=1)
        post_h = jnp.concatenate([x, agg, agg * s2c, agg * s3c], axis=1)
        o_parts.append(_dot(post_h, wpost_ref[t])
                       + bpost_ref[:, t * _FO:(t + 1) * _FO])
    y = _dot(jnp.concatenate(o_parts, axis=1), wlin_ref[...]) + blin_ref[...]
    y_ref[...] = y

    @pl.when(i == 0)
    def _():
        bnsum_ref[...] = jnp.zeros_like(bnsum_ref)
        bnssq_ref[...] = jnp.zeros_like(bnssq_ref)

    bnsum_ref[...] += jnp.sum(y, axis=0, keepdims=True)
    bnssq_ref[...] += jnp.sum(y * y, axis=0, keepdims=True)


def _k_post(x, P, S1, S2, MN, MX, deg2d, Wpost_l, bpost_f, Wlin_l, blin_f):
    nb = 200
    big = lambda: pl.BlockSpec((nb, _T * _F), lambda i: (i, 0))
    return pl.pallas_call(
        _kpost_body,
        grid=(_N // nb,),
        in_specs=[
            pl.BlockSpec((nb, _D), lambda i: (i, 0)),
            big(), big(), big(), big(), big(),
            pl.BlockSpec((nb, 1), lambda i: (i, 0)),
            pl.BlockSpec((_T, 13 * _F, _FO), lambda i: (0, 0, 0)),
            pl.BlockSpec((1, _T * _FO), lambda i: (0, 0)),
            pl.BlockSpec((_D, _D), lambda i: (0, 0)),
            pl.BlockSpec((1, _D), lambda i: (0, 0)),
        ],
        out_specs=[
            pl.BlockSpec((nb, _D), lambda i: (i, 0)),
            pl.BlockSpec((1, _D), lambda i: (0, 0)),
            pl.BlockSpec((1, _D), lambda i: (0, 0)),
        ],
        out_shape=[
            jax.ShapeDtypeStruct((_N, _D), jnp.float32),
            jax.ShapeDtypeStruct((1, _D), jnp.float32),
            jax.ShapeDtypeStruct((1, _D), jnp.float32),
        ],
    )(x, P, S1, S2, MN, MX, deg2d, Wpost_l, bpost_f, Wlin_l, blin_f)


def _kbn_body(y_ref, sum_ref, ssq_ref, gamma_ref, beta_ref, o_ref):
    m = sum_ref[...] * (1.0 / _N)
    var = ssq_ref[...] * (1.0 / _N) - m * m
    inv = jax.lax.rsqrt(var + 1e-5)
    o_ref[...] = jnp.maximum((y_ref[...] - m) * inv * gamma_ref[...]
                             + beta_ref[...], 0.0)


def _k_bn(y, bnsum, bnssq, gamma_f, beta_f):
    nb = 1000
    return pl.pallas_call(
        _kbn_body,
        grid=(_N // nb,),
        in_specs=[
            pl.BlockSpec((nb, _D), lambda i: (i, 0)),
            pl.BlockSpec((1, _D), lambda i: (0, 0)),
            pl.BlockSpec((1, _D), lambda i: (0, 0)),
            pl.BlockSpec((1, _D), lambda i: (0, 0)),
            pl.BlockSpec((1, _D), lambda i: (0, 0)),
        ],
        out_specs=pl.BlockSpec((nb, _D), lambda i: (i, 0)),
        out_shape=jax.ShapeDtypeStruct((_N, _D), jnp.float32),
    )(y, bnsum, bnssq, gamma_f, beta_f)


def _kpool_body(x_ref, b_ref, o_ref):
    i = pl.program_id(0)

    @pl.when(i == 0)
    def _():
        o_ref[...] = jnp.zeros_like(o_ref)

    b = b_ref[...]
    gid = jax.lax.broadcasted_iota(jnp.int32, (b.shape[0], _NG), 1)
    oh = (b == gid).astype(jnp.float32)
    o_ref[...] += jax.lax.dot_general(oh, x_ref[...], (((0,), (0,)), ((), ())),
                                      precision=_HI,
                                      preferred_element_type=jnp.float32)


def _k_pool(x, batch2d):
    nb = 1000
    return pl.pallas_call(
        _kpool_body,
        grid=(_N // nb,),
        in_specs=[
            pl.BlockSpec((nb, _D), lambda i: (i, 0)),
            pl.BlockSpec((nb, 1), lambda i: (i, 0)),
        ],
        out_specs=pl.BlockSpec((_NG, _D), lambda i: (0, 0)),
        out_shape=jax.ShapeDtypeStruct((_NG, _D), jnp.float32),
    )(x, batch2d)


def _khead_body(p_ref, wf_ref, bf_ref, wo_ref, bo_ref, a_ref, o_ref):
    h = _dot(p_ref[...], wf_ref[...]) + bf_ref[...]
    a = a_ref[0, 0]
    h = jnp.where(h >= 0.0, h, a * h)
    logits = _dot(h, wo_ref[...]) + bo_ref[...]
    mx = jnp.max(logits, axis=1, keepdims=True)
    lse = jnp.log(jnp.sum(jnp.exp(logits - mx), axis=1, keepdims=True)) + mx
    o_ref[...] = logits - lse


def _k_head(pooled, W_fc1, b_fc1, W_out, b_out, prelu_a):
    return pl.pallas_call(
        _khead_body,
        out_shape=jax.ShapeDtypeStruct((_NG, 2), jnp.float32),
    )(pooled, W_fc1, b_fc1.reshape(1, 2 * _D), W_out, b_out.reshape(1, 2),
      prelu_a.reshape(1, 1).astype(jnp.float32))


def _embed_gather(node_emb, global_idx):
    return node_emb[global_idx]


def _seg_reduce(Q, C2, src, dst, sign):
    m = Q[src] + C2[sign]
    deg = jax.ops.segment_sum(jnp.ones((_E,), jnp.float32), dst,
                              num_segments=_N)
    S1 = jax.ops.segment_sum(m, dst, num_segments=_N)
    S2 = jax.ops.segment_sum(m * m, dst, num_segments=_N)
    MN = jax.ops.segment_min(m, dst, num_segments=_N)
    MX = jax.ops.segment_max(m, dst, num_segments=_N)
    return S1, S2, MN, MX, deg


def kernel(global_idx, acts, sign, edge_index, batch, node_emb, edge_emb,
           W_act, b_act, We, be, Wpre, bpre, Wpost, bpost, Wlin, blin,
           gamma, beta, W_fc1, b_fc1, W_out, b_out, prelu_a):
    src = edge_index[0].astype(jnp.int32)
    dst = edge_index[1].astype(jnp.int32)
    sgn = sign.astype(jnp.int32)

    xe = _embed_gather(node_emb, global_idx.astype(jnp.int32))
    x = _k_x(xe, acts, W_act, b_act.reshape(1, _D))

    for l in range(2):
        Wd = Wpre[l][:, :_F, :].transpose(1, 0, 2).reshape(_F, _T * _F)
        Ws = Wpre[l][:, _F:2 * _F, :].transpose(1, 0, 2).reshape(_F, _T * _F)
        Wpre_e = Wpre[l][:, 2 * _F:, :]
        bpre_f = bpre[l].reshape(1, _T * _F)
        C2 = _k_c2(edge_emb, We[l], be[l], Wpre_e)
        P, Q = _k_pre(x, Wd, Ws, bpre_f)
        S1, S2, MN, MX, deg = _seg_reduce(Q, C2, src, dst, sgn)
        y, bnsum, bnssq = _k_post(x, P, S1, S2, MN, MX, deg.reshape(-1, 1),
                                  Wpost[l], bpost[l].reshape(1, _T * _FO),
                                  Wlin[l], blin[l].reshape(1, _D))
        x = _k_bn(y, bnsum, bnssq, gamma[l].reshape(1, _D),
                  beta[l].reshape(1, _D))

    pooled = _k_pool(x, batch.astype(jnp.int32).reshape(-1, 1))
    return _k_head(pooled, W_fc1, b_fc1, W_out, b_out, prelu_a)
